# Initial kernel scaffold; baseline (speedup 1.0000x reference)
import jax
import jax.numpy as jnp
from jax.experimental import pallas as pl


def kernel(label_emb, w1_cat, w2_bd):
    raise NotImplementedError("write your pallas kernel here")


# speedup vs baseline: 2.1302x; 2.1302x over previous
import jax
import jax.numpy as jnp
from jax.experimental import pallas as pl
from jax.experimental.pallas import tpu as pltpu


def _packed_ffn_kernel(x_ref, w1_ref, w2_ref, o_ref):
    w1 = w1_ref[...].astype(jnp.bfloat16)
    w2 = w2_ref[...].astype(jnp.bfloat16)
    z1 = jnp.zeros_like(w1)
    z2 = jnp.zeros_like(w2)
    w1p = jnp.concatenate(
        [jnp.concatenate([w1, z1], axis=1), jnp.concatenate([z1, w1], axis=1)],
        axis=0,
    )
    w2p = jnp.concatenate(
        [jnp.concatenate([w2, z2], axis=1), jnp.concatenate([z2, w2], axis=1)],
        axis=0,
    )
    x = x_ref[...].astype(jnp.bfloat16)
    h = jnp.tanh(jnp.dot(x, w1p, preferred_element_type=jnp.float32))
    y = jnp.dot(h.astype(jnp.bfloat16), w2p, preferred_element_type=jnp.float32)
    o_ref[...] = y


def kernel(label_emb, w1_cat, w2_bd):
    L, E = label_emb.shape
    HE = w1_cat.shape[1]

    if L % 2 == 0:
        pack, M = 2, L // 2
    else:
        pack, M = 1, L
    xp = label_emb.reshape(M, pack * E)

    T = 8192
    while M % T:
        T //= 2
    grid = M // T

    out = pl.pallas_call(
        _packed_ffn_kernel if pack == 2 else _unpacked_ffn_kernel,
        out_shape=jax.ShapeDtypeStruct((M, pack * HE), jnp.float32),
        grid=(grid,),
        in_specs=[
            pl.BlockSpec((T, pack * E), lambda i: (i, 0)),
            pl.BlockSpec((E, HE), lambda i: (0, 0)),
            pl.BlockSpec((HE, HE), lambda i: (0, 0)),
        ],
        out_specs=pl.BlockSpec((T, pack * HE), lambda i: (i, 0)),
        compiler_params=pltpu.CompilerParams(dimension_semantics=("parallel",)),
        cost_estimate=pl.CostEstimate(
            flops=2 * L * E * HE + 2 * L * HE * HE,
            transcendentals=L * HE,
            bytes_accessed=(L * E + L * HE) * 4 + (E * HE + HE * HE) * 4,
        ),
    )(xp, w1_cat, w2_bd)
    return out.reshape(L, HE).astype(label_emb.dtype)


def _unpacked_ffn_kernel(x_ref, w1_ref, w2_ref, o_ref):
    w1 = w1_ref[...].astype(jnp.bfloat16)
    w2 = w2_ref[...].astype(jnp.bfloat16)
    x = x_ref[...].astype(jnp.bfloat16)
    h = jnp.tanh(jnp.dot(x, w1, preferred_element_type=jnp.float32))
    o_ref[...] = jnp.dot(h.astype(jnp.bfloat16), w2, preferred_element_type=jnp.float32)


# speedup vs baseline: 4.2767x; 2.0076x over previous
import jax
import jax.numpy as jnp
from jax.experimental import pallas as pl
from jax.experimental.pallas import tpu as pltpu


def _packed_ffn_kernel(x_ref, w1_ref, w2_ref, o_ref):
    R = x_ref.shape[0]
    w1 = w1_ref[...].astype(jnp.bfloat16)
    w2 = w2_ref[...].astype(jnp.bfloat16)
    z2 = jnp.zeros_like(w2)
    w2p = jnp.concatenate(
        [jnp.concatenate([w2, z2], axis=1), jnp.concatenate([z2, w2], axis=1)],
        axis=0,
    )
    x = x_ref[...].astype(jnp.bfloat16)
    h = jnp.tanh(jnp.dot(x, w1, preferred_element_type=jnp.float32))
    hp = jnp.concatenate(
        [h[: R // 2].astype(jnp.bfloat16), h[R // 2 :].astype(jnp.bfloat16)],
        axis=1,
    )
    y = jnp.dot(hp, w2p, preferred_element_type=jnp.float32)
    o_ref[: R // 2, :] = y[:, :128]
    o_ref[R // 2 :, :] = y[:, 128:]


def kernel(label_emb, w1_cat, w2_bd):
    L, E = label_emb.shape
    HE = w1_cat.shape[1]

    R = 8192
    while L % R:
        R //= 2

    out = pl.pallas_call(
        _packed_ffn_kernel if R % 2 == 0 else _unpacked_ffn_kernel,
        out_shape=jax.ShapeDtypeStruct((L, HE), jnp.float32),
        grid=(L // R,),
        in_specs=[
            pl.BlockSpec((R, E), lambda i: (i, 0)),
            pl.BlockSpec((E, HE), lambda i: (0, 0)),
            pl.BlockSpec((HE, HE), lambda i: (0, 0)),
        ],
        out_specs=pl.BlockSpec((R, HE), lambda i: (i, 0)),
        compiler_params=pltpu.CompilerParams(dimension_semantics=("parallel",)),
        cost_estimate=pl.CostEstimate(
            flops=2 * L * E * HE + 2 * L * HE * HE,
            transcendentals=L * HE,
            bytes_accessed=(L * E + L * HE) * 4 + (E * HE + HE * HE) * 4,
        ),
    )(label_emb, w1_cat, w2_bd)
    return out.astype(label_emb.dtype)


def _unpacked_ffn_kernel(x_ref, w1_ref, w2_ref, o_ref):
    w1 = w1_ref[...].astype(jnp.bfloat16)
    w2 = w2_ref[...].astype(jnp.bfloat16)
    x = x_ref[...].astype(jnp.bfloat16)
    h = jnp.tanh(jnp.dot(x, w1, preferred_element_type=jnp.float32))
    o_ref[...] = jnp.dot(h.astype(jnp.bfloat16), w2, preferred_element_type=jnp.float32)


# speedup vs baseline: 4.2949x; 1.0042x over previous
import jax
import jax.numpy as jnp
from jax.experimental import pallas as pl
from jax.experimental.pallas import tpu as pltpu


def _packed_ffn_kernel(x_ref, w1_ref, w2_ref, o_ref):
    R = x_ref.shape[0]
    w1 = w1_ref[...].astype(jnp.bfloat16)
    w2 = w2_ref[...].astype(jnp.bfloat16)
    z2 = jnp.zeros_like(w2)
    w2p = jnp.concatenate(
        [jnp.concatenate([w2, z2], axis=1), jnp.concatenate([z2, w2], axis=1)],
        axis=0,
    )
    x = x_ref[...].astype(jnp.bfloat16)
    h = jnp.tanh(jnp.dot(x, w1, preferred_element_type=jnp.float32))
    hp = jnp.concatenate(
        [h[: R // 2].astype(jnp.bfloat16), h[R // 2 :].astype(jnp.bfloat16)],
        axis=1,
    )
    y = jnp.dot(hp, w2p, preferred_element_type=jnp.float32)
    o_ref[: R // 2, :] = y[:, :128]
    o_ref[R // 2 :, :] = y[:, 128:]


def kernel(label_emb, w1_cat, w2_bd):
    L, E = label_emb.shape
    HE = w1_cat.shape[1]

    R = 8192
    while L % R:
        R //= 2

    out = pl.pallas_call(
        _packed_ffn_kernel if R % 2 == 0 else _unpacked_ffn_kernel,
        out_shape=jax.ShapeDtypeStruct((L, HE), label_emb.dtype),
        grid=(L // R,),
        in_specs=[
            pl.BlockSpec((R, E), lambda i: (i, 0)),
            pl.BlockSpec((E, HE), lambda i: (0, 0)),
            pl.BlockSpec((HE, HE), lambda i: (0, 0)),
        ],
        out_specs=pl.BlockSpec((R, HE), lambda i: (i, 0)),
        compiler_params=pltpu.CompilerParams(dimension_semantics=("parallel",)),
        cost_estimate=pl.CostEstimate(
            flops=2 * L * E * HE + 2 * L * HE * HE,
            transcendentals=L * HE,
            bytes_accessed=(L * E + L * HE) * 4 + (E * HE + HE * HE) * 4,
        ),
    )(label_emb, w1_cat, w2_bd)
    return out


def _unpacked_ffn_kernel(x_ref, w1_ref, w2_ref, o_ref):
    w1 = w1_ref[...].astype(jnp.bfloat16)
    w2 = w2_ref[...].astype(jnp.bfloat16)
    x = x_ref[...].astype(jnp.bfloat16)
    h = jnp.tanh(jnp.dot(x, w1, preferred_element_type=jnp.float32))
    o_ref[...] = jnp.dot(h.astype(jnp.bfloat16), w2, preferred_element_type=jnp.float32)


# speedup vs baseline: 4.4850x; 1.0443x over previous
import jax
import jax.numpy as jnp
from jax.experimental import pallas as pl
from jax.experimental.pallas import tpu as pltpu


def _packed_ffn_kernel(x_ref, w1_ref, w2_ref, o_ref):
    R = x_ref.shape[0]
    w1 = w1_ref[...].astype(jnp.bfloat16)
    w2 = w2_ref[...].astype(jnp.bfloat16)
    z2 = jnp.zeros_like(w2)
    w2p = jnp.concatenate(
        [jnp.concatenate([w2, z2], axis=1), jnp.concatenate([z2, w2], axis=1)],
        axis=0,
    )
    x = x_ref[...].astype(jnp.bfloat16)
    h = jnp.tanh(jnp.dot(x, w1, preferred_element_type=jnp.float32))
    hp = jnp.concatenate(
        [h[: R // 2].astype(jnp.bfloat16), h[R // 2 :].astype(jnp.bfloat16)],
        axis=1,
    )
    y = jnp.dot(hp, w2p, preferred_element_type=jnp.float32)
    o_ref[: R // 2, :] = y[:, :128]
    o_ref[R // 2 :, :] = y[:, 128:]


def kernel(label_emb, w1_cat, w2_bd):
    L, E = label_emb.shape
    HE = w1_cat.shape[1]

    R = 16384
    while L % R:
        R //= 2

    out = pl.pallas_call(
        _packed_ffn_kernel if R % 2 == 0 else _unpacked_ffn_kernel,
        out_shape=jax.ShapeDtypeStruct((L, HE), label_emb.dtype),
        grid=(L // R,),
        in_specs=[
            pl.BlockSpec((R, E), lambda i: (i, 0)),
            pl.BlockSpec((E, HE), lambda i: (0, 0)),
            pl.BlockSpec((HE, HE), lambda i: (0, 0)),
        ],
        out_specs=pl.BlockSpec((R, HE), lambda i: (i, 0)),
        compiler_params=pltpu.CompilerParams(dimension_semantics=("parallel",)),
        cost_estimate=pl.CostEstimate(
            flops=2 * L * E * HE + 2 * L * HE * HE,
            transcendentals=L * HE,
            bytes_accessed=(L * E + L * HE) * 4 + (E * HE + HE * HE) * 4,
        ),
    )(label_emb, w1_cat, w2_bd)
    return out


def _unpacked_ffn_kernel(x_ref, w1_ref, w2_ref, o_ref):
    w1 = w1_ref[...].astype(jnp.bfloat16)
    w2 = w2_ref[...].astype(jnp.bfloat16)
    x = x_ref[...].astype(jnp.bfloat16)
    h = jnp.tanh(jnp.dot(x, w1, preferred_element_type=jnp.float32))
    o_ref[...] = jnp.dot(h.astype(jnp.bfloat16), w2, preferred_element_type=jnp.float32)
